# baseline (speedup 1.0000x reference)
import functools

import jax
import jax.numpy as jnp
from jax import lax
from jax.experimental import pallas as pl
from jax.experimental.pallas import tpu as pltpu
from jax.experimental.pallas import tpu_sc as plsc

NUM_WORKERS = 32
BC = 128
LANES = 16


def _emb_lookup(table, idxT, H, NBT, D):
    mesh = plsc.VectorSubcoreMesh(core_axis_name="c", subcore_axis_name="s")
    n_tasks = H * NBT
    tpw = n_tasks // NUM_WORKERS
    DT = D // 8

    @functools.partial(
        pl.kernel,
        mesh=mesh,
        out_type=jax.ShapeDtypeStruct((H, DT, NBT, 8, BC), jnp.float32),
        scratch_types=[
            pltpu.VMEM((tpw, BC), jnp.int32),
            pltpu.VMEM((BC, D), jnp.float32),
            pltpu.VMEM((BC, D), jnp.float32),
            pltpu.VMEM((D, BC), jnp.float32),
            pltpu.VMEM((D, BC), jnp.float32),
            pltpu.SemaphoreType.DMA((2,)),
            pltpu.SemaphoreType.DMA((2,)),
        ],
        compiler_params=pltpu.CompilerParams(
            use_tc_tiling_on_sc=False, needs_layout_passes=False
        ),
    )
    def emb(table_hbm, idx_hbm, out_hbm, idx_v, rows0, rows1, tr0, tr1,
            sem_g, sem_s):
        wid = lax.axis_index("s") * 2 + lax.axis_index("c")
        t0 = wid * tpw
        pltpu.sync_copy(idx_hbm.at[pl.ds(t0, tpw)], idx_v)

        lane = lax.iota(jnp.int32, LANES)

        def start_gather(t, rows, b):
            pltpu.make_async_copy(
                table_hbm.at[idx_v.at[t]], rows, sem_g.at[b]
            ).start()

        def wait_gather(rows, b):
            pltpu.make_async_copy(
                table_hbm.at[idx_v.at[0]], rows, sem_g.at[b]
            ).wait()

        def transpose(rows, tr):
            for bg in range(BC // LANES):
                rid = lane + (bg * LANES)
                for dd in range(D):
                    cid = jnp.full((LANES,), dd, jnp.int32)
                    tr[dd, pl.ds(bg * LANES, LANES)] = plsc.load_gather(
                        rows, [rid, cid]
                    )

        def start_scatter(t, tr, b):
            tg = t0 + t
            h = tg // NBT
            bt = tg % NBT
            for dt in range(DT):
                pltpu.make_async_copy(
                    tr.at[pl.ds(dt * 8, 8)], out_hbm.at[h, dt, bt],
                    sem_s.at[b],
                ).start()

        def wait_scatter(tr, b):
            for dt in range(DT):
                pltpu.make_async_copy(
                    tr.at[pl.ds(0, 8)], out_hbm.at[0, 0, 0], sem_s.at[b]
                ).wait()

        start_gather(0, rows0, 0)
        start_gather(1, rows1, 1)

        def group(g, carry):
            for b, rows, tr in ((0, rows0, tr0), (1, rows1, tr1)):
                t = g * 2 + b
                wait_gather(rows, b)

                @pl.when(g >= 1)
                def _():
                    wait_scatter(tr, b)

                transpose(rows, tr)
                start_scatter(t, tr, b)
                start_gather(jnp.minimum(t + 2, tpw - 1), rows, b)
            return carry

        lax.fori_loop(0, tpw // 2, group, 0)
        wait_gather(rows0, 0)
        wait_gather(rows1, 1)
        wait_scatter(tr0, 0)
        wait_scatter(tr1, 1)

    return emb(table, idxT)


def kernel(x, item_emb_weight):
    B, H = x.shape
    V, D = item_emb_weight.shape
    NBT = B // BC
    idxT = jnp.transpose(x).reshape(H * NBT, BC).astype(jnp.int32)
    Y = _emb_lookup(item_emb_weight, idxT, H, NBT, D)
    return Y.transpose(2, 4, 0, 1, 3).reshape(B, H, D)


# speedup vs baseline: 1.5653x; 1.5653x over previous
import functools

import jax
import jax.numpy as jnp
from jax import lax
from jax.experimental import pallas as pl
from jax.experimental.pallas import tpu as pltpu
from jax.experimental.pallas import tpu_sc as plsc

NUM_WORKERS = 32
BC = 128
LANES = 16


def _emb_lookup(table, idxT, H, NBT, D):
    mesh = plsc.VectorSubcoreMesh(core_axis_name="c", subcore_axis_name="s")
    n_tasks = H * NBT
    tpw = n_tasks // NUM_WORKERS
    DT = D // 8

    @functools.partial(
        pl.kernel,
        mesh=mesh,
        out_type=jax.ShapeDtypeStruct((H, DT, NBT, 8, BC), jnp.float32),
        scratch_types=[
            pltpu.VMEM((tpw, BC), jnp.int32),
            pltpu.VMEM((BC, D), jnp.float32),
            pltpu.VMEM((BC, D), jnp.float32),
            pltpu.VMEM((D, BC), jnp.float32),
            pltpu.VMEM((D, BC), jnp.float32),
            pltpu.SemaphoreType.DMA((2,)),
            pltpu.SemaphoreType.DMA((2,)),
        ],
        compiler_params=pltpu.CompilerParams(
            use_tc_tiling_on_sc=False, needs_layout_passes=False
        ),
    )
    def emb(table_hbm, idx_hbm, out_hbm, idx_v, rows0, rows1, tr0, tr1,
            sem_g, sem_s):
        wid = lax.axis_index("s") * 2 + lax.axis_index("c")
        t0 = wid * tpw
        pltpu.sync_copy(idx_hbm.at[pl.ds(t0, tpw)], idx_v)

        lane = lax.iota(jnp.int32, LANES)

        def start_gather(t, rows, b):
            pltpu.make_async_copy(
                table_hbm.at[idx_v.at[t]], rows, sem_g.at[b]
            ).start()

        def wait_gather(rows, b):
            pltpu.make_async_copy(
                table_hbm.at[idx_v.at[0]], rows, sem_g.at[b]
            ).wait()

        def transpose(rows, tr):
            @plsc.parallel_loop(0, D, unroll=8)
            def _(dd):
                cid = jnp.full((LANES,), 0, jnp.int32) + dd
                for bg in range(BC // LANES):
                    rid = lane + (bg * LANES)
                    tr[dd, pl.ds(bg * LANES, LANES)] = plsc.load_gather(
                        rows, [rid, cid]
                    )

        def start_scatter(t, tr, b):
            tg = t0 + t
            h = tg // NBT
            bt = tg % NBT
            for dt in range(DT):
                pltpu.make_async_copy(
                    tr.at[pl.ds(dt * 8, 8)], out_hbm.at[h, dt, bt],
                    sem_s.at[b],
                ).start()

        def wait_scatter(tr, b):
            for dt in range(DT):
                pltpu.make_async_copy(
                    tr.at[pl.ds(0, 8)], out_hbm.at[0, 0, 0], sem_s.at[b]
                ).wait()

        start_gather(0, rows0, 0)
        start_gather(1, rows1, 1)

        def group(g, carry):
            for b, rows, tr in ((0, rows0, tr0), (1, rows1, tr1)):
                t = g * 2 + b
                wait_gather(rows, b)

                @pl.when(g >= 1)
                def _():
                    wait_scatter(tr, b)

                transpose(rows, tr)
                start_scatter(t, tr, b)
                start_gather(jnp.minimum(t + 2, tpw - 1), rows, b)
            return carry

        lax.fori_loop(0, tpw // 2, group, 0)
        wait_gather(rows0, 0)
        wait_gather(rows1, 1)
        wait_scatter(tr0, 0)
        wait_scatter(tr1, 1)

    return emb(table, idxT)


def kernel(x, item_emb_weight):
    B, H = x.shape
    V, D = item_emb_weight.shape
    NBT = B // BC
    idxT = jnp.transpose(x).reshape(H * NBT, BC).astype(jnp.int32)
    Y = _emb_lookup(item_emb_weight, idxT, H, NBT, D)
    return Y.transpose(2, 4, 0, 1, 3).reshape(B, H, D)
